# baseline (speedup 1.0000x reference)
import functools

import jax
import jax.numpy as jnp
from jax import lax
from jax.experimental import pallas as pl
from jax.experimental.pallas import tpu as pltpu
from jax.experimental.pallas import tpu_sc as plsc

DIM = 128
NC = 2
NS = 16
NW = NC * NS
B_TOTAL = 4 * 8192
CHUNK = 128
N_CHUNKS = 7
B_PER_W = N_CHUNKS * CHUNK
B_SC = NW * B_PER_W
B_TC = B_TOTAL - B_SC

NBUF = 6

_mesh = plsc.VectorSubcoreMesh(core_axis_name="c", subcore_axis_name="s")


@functools.partial(
    pl.kernel,
    mesh=_mesh,
    out_type=jax.ShapeDtypeStruct((2, B_TOTAL, DIM), jnp.float32),
    scratch_types=[
        pltpu.VMEM((N_CHUNKS, CHUNK), jnp.int32),
        *([pltpu.VMEM((CHUNK, DIM), jnp.float32)] * NBUF),
        *([pltpu.SemaphoreType.DMA] * (2 * NBUF)),
    ],
)
def _rope_gather(cos_hbm, sin_hbm, idx_hbm, out_hbm, idx_v, *bufs_and_sems):
    bufs = bufs_and_sems[:NBUF]
    gsems = bufs_and_sems[NBUF:2 * NBUF]
    ssems = bufs_and_sems[2 * NBUF:]
    wid = lax.axis_index("s") * NC + lax.axis_index("c")
    base = wid * B_PER_W
    pltpu.sync_copy(idx_hbm.at[wid], idx_v)
    tables = (cos_hbm, sin_hbm)
    items = [(t, c) for c in range(N_CHUNKS) for t in range(2)]
    n = len(items)
    gd = [None] * NBUF
    sd = [None] * NBUF
    for j in range(min(NBUF - 1, n)):
        tj, cj = items[j]
        gd[j] = pltpu.async_copy(tables[tj].at[idx_v.at[cj]], bufs[j], gsems[j])
    for i in range(n):
        b = i % NBUF
        j = i + NBUF - 1
        if j < n:
            jb = j % NBUF
            if sd[jb] is not None:
                sd[jb].wait()
            tj, cj = items[j]
            gd[jb] = pltpu.async_copy(tables[tj].at[idx_v.at[cj]], bufs[jb], gsems[jb])
        gd[b].wait()
        t, c = items[i]
        sd[b] = pltpu.async_copy(bufs[b], out_hbm.at[t, pl.ds(base + c * CHUNK, CHUNK)], ssems[b])
    for b in range(NBUF):
        if sd[b] is not None:
            sd[b].wait()


def kernel(cos_cached, sin_cached, position_ids):
    flat = position_ids.reshape(-1)
    idx_sc = flat[:B_SC].reshape(NW, N_CHUNKS, CHUNK)
    out = _rope_gather(cos_cached, sin_cached, idx_sc)
    idx_tc = flat[B_SC:]
    tc_part = jnp.stack([jnp.take(cos_cached, idx_tc, axis=0),
                         jnp.take(sin_cached, idx_tc, axis=0)])
    out = lax.dynamic_update_slice(out, tc_part, (0, B_SC, 0))
    return out.reshape(2, 4, 8192, DIM)


# speedup vs baseline: 1.1859x; 1.1859x over previous
import functools

import jax
import jax.numpy as jnp
from jax import lax
from jax.experimental import pallas as pl
from jax.experimental.pallas import tpu as pltpu
from jax.experimental.pallas import tpu_sc as plsc

DIM = 128
BASE_CONST = 10000
NC = 2
NS = 16
NW = NC * NS
B_TOTAL = 4 * 8192
CHUNK = 128
N_CHUNKS = 7
B_PER_W = N_CHUNKS * CHUNK
B_SC = NW * B_PER_W
B_TC = B_TOTAL - B_SC

NBUF = 6

_mesh = plsc.VectorSubcoreMesh(core_axis_name="c", subcore_axis_name="s")


@functools.partial(
    pl.kernel,
    mesh=_mesh,
    out_type=jax.ShapeDtypeStruct((2, B_TOTAL, DIM), jnp.float32),
    scratch_types=[
        pltpu.VMEM((N_CHUNKS, CHUNK), jnp.int32),
        *([pltpu.VMEM((CHUNK, DIM), jnp.float32)] * NBUF),
        *([pltpu.SemaphoreType.DMA] * (2 * NBUF)),
    ],
)
def _rope_gather(cos_hbm, sin_hbm, idx_hbm, out_hbm, idx_v, *bufs_and_sems):
    bufs = bufs_and_sems[:NBUF]
    gsems = bufs_and_sems[NBUF:2 * NBUF]
    ssems = bufs_and_sems[2 * NBUF:]
    wid = lax.axis_index("s") * NC + lax.axis_index("c")
    base = wid * B_PER_W
    pltpu.sync_copy(idx_hbm.at[wid], idx_v)
    tables = (cos_hbm, sin_hbm)
    items = [(t, c) for c in range(N_CHUNKS) for t in range(2)]
    n = len(items)
    gd = [None] * NBUF
    sd = [None] * NBUF
    for j in range(min(NBUF - 1, n)):
        tj, cj = items[j]
        gd[j] = pltpu.async_copy(tables[tj].at[idx_v.at[cj]], bufs[j], gsems[j])
    for i in range(n):
        b = i % NBUF
        j = i + NBUF - 1
        if j < n:
            jb = j % NBUF
            if sd[jb] is not None:
                sd[jb].wait()
            tj, cj = items[j]
            gd[jb] = pltpu.async_copy(tables[tj].at[idx_v.at[cj]], bufs[jb], gsems[jb])
        gd[b].wait()
        t, c = items[i]
        sd[b] = pltpu.async_copy(bufs[b], out_hbm.at[t, pl.ds(base + c * CHUNK, CHUNK)], ssems[b])
    for b in range(NBUF):
        if sd[b] is not None:
            sd[b].wait()


def _tail_rope_body(idx_ref, f_ref, out_ref):
    pos = idx_ref[:].astype(jnp.float32)
    ang = pos * f_ref[:]
    ang2 = jnp.concatenate([ang, ang], axis=-1)
    out_ref[0] = jnp.cos(ang2)
    out_ref[1] = jnp.sin(ang2)


_tail_rope = pl.pallas_call(
    _tail_rope_body,
    out_shape=jax.ShapeDtypeStruct((2, B_TC, DIM), jnp.float32),
)


def kernel(cos_cached, sin_cached, position_ids):
    flat = position_ids.reshape(-1)
    idx_sc = flat[:B_SC].reshape(NW, N_CHUNKS, CHUNK)
    out = _rope_gather(cos_cached, sin_cached, idx_sc)
    inv_freq = 1.0 / (BASE_CONST ** (jnp.arange(0, DIM, 2, dtype=jnp.float32) / DIM))
    tc_part = _tail_rope(flat[B_SC:].reshape(B_TC, 1), inv_freq.reshape(1, DIM // 2))
    out = lax.dynamic_update_slice(out, tc_part, (0, B_SC, 0))
    return out.reshape(2, 4, 8192, DIM)


# speedup vs baseline: 1.2699x; 1.0709x over previous
import functools

import jax
import jax.numpy as jnp
from jax import lax
from jax.experimental import pallas as pl
from jax.experimental.pallas import tpu as pltpu
from jax.experimental.pallas import tpu_sc as plsc

DIM = 128
NC = 2
NS = 16
NW = NC * NS
B_TOTAL = 4 * 8192
B_PER_W = B_TOTAL // NW
CHUNK = 128
N_CHUNKS = B_PER_W // CHUNK

NBUF = 6

_mesh = plsc.VectorSubcoreMesh(core_axis_name="c", subcore_axis_name="s")


@functools.partial(
    pl.kernel,
    mesh=_mesh,
    out_type=jax.ShapeDtypeStruct((2, B_TOTAL, DIM), jnp.float32),
    scratch_types=[
        pltpu.VMEM((N_CHUNKS, CHUNK), jnp.int32),
        *([pltpu.VMEM((CHUNK, DIM), jnp.float32)] * NBUF),
        *([pltpu.SemaphoreType.DMA] * (2 * NBUF)),
    ],
)
def _rope_gather(cos_hbm, sin_hbm, idx_hbm, out_hbm, idx_v, *bufs_and_sems):
    bufs = bufs_and_sems[:NBUF]
    gsems = bufs_and_sems[NBUF:2 * NBUF]
    ssems = bufs_and_sems[2 * NBUF:]
    wid = lax.axis_index("s") * NC + lax.axis_index("c")
    base = wid * B_PER_W
    pltpu.sync_copy(idx_hbm.at[wid // 8, pl.ds((wid % 8) * 8, N_CHUNKS)], idx_v)
    tables = (cos_hbm, sin_hbm)
    items = [(t, c) for c in range(N_CHUNKS) for t in range(2)]
    n = len(items)
    gd = [None] * NBUF
    sd = [None] * NBUF
    for j in range(min(NBUF - 1, n)):
        tj, cj = items[j]
        gd[j] = pltpu.async_copy(tables[tj].at[idx_v.at[cj]], bufs[j], gsems[j])
    for i in range(n):
        b = i % NBUF
        j = i + NBUF - 1
        if j < n:
            jb = j % NBUF
            if sd[jb] is not None:
                sd[jb].wait()
            tj, cj = items[j]
            gd[jb] = pltpu.async_copy(tables[tj].at[idx_v.at[cj]], bufs[jb], gsems[jb])
        gd[b].wait()
        t, c = items[i]
        sd[b] = pltpu.async_copy(bufs[b], out_hbm.at[t, pl.ds(base + c * CHUNK, CHUNK)], ssems[b])
    for b in range(NBUF):
        if sd[b] is not None:
            sd[b].wait()


def kernel(cos_cached, sin_cached, position_ids):
    idx = position_ids.reshape(4, 64, CHUNK)
    out = _rope_gather(cos_cached, sin_cached, idx)
    return out.reshape(2, 4, 8192, DIM)
